# Initial kernel scaffold; baseline (speedup 1.0000x reference)
import jax
import jax.numpy as jnp
from jax.experimental import pallas as pl


def kernel(x, pos, H, W):
    raise NotImplementedError("write your pallas kernel here")


# speedup vs baseline: 1762.9530x; 1762.9530x over previous
import functools

import jax
import jax.numpy as jnp
from jax import lax
from jax.experimental import pallas as pl
from jax.experimental.pallas import tpu as pltpu
from jax.experimental.pallas import tpu_sc as plsc

_NC = 2
_NS = 16
_NW = _NC * _NS
_L = 16
_K = 80


def _interp_sc(tbl, pxs, pys, B, C, Hx, Wx, N):
    P = B * N
    n_chunks = P // _K
    cpb = N // _K
    base_per_w = n_chunks // _NW
    rem = n_chunks - base_per_w * _NW

    mesh = plsc.VectorSubcoreMesh(
        core_axis_name="c", subcore_axis_name="s",
        num_cores=_NC, num_subcores=_NS)

    @functools.partial(
        pl.kernel,
        out_type=jax.ShapeDtypeStruct((P, C), jnp.float32),
        mesh=mesh,
        scratch_types=[
            pltpu.VMEM((_K,), jnp.float32),
            pltpu.VMEM((_K,), jnp.float32),
            pltpu.VMEM((4, _K), jnp.int32),
            pltpu.VMEM((4, _K), jnp.float32),
            pltpu.VMEM((4, _K, C), jnp.float32),
            pltpu.VMEM((_K, C), jnp.float32),
            pltpu.SemaphoreType.DMA,
        ],
        compiler_params=pltpu.CompilerParams(use_tc_tiling_on_sc=False),
    )
    def k(tbl_ref, px_ref, py_ref, out_ref, pxv, pyv, idx4, w4, bufs, acc, sem):
        wid = lax.axis_index("s") * _NC + lax.axis_index("c")
        n_w = base_per_w + (wid < rem).astype(jnp.int32)

        def chunk_body(t, carry):
            cid = wid + t * _NW
            cbase = cid * _K
            tb = (cid // cpb) * (Hx * Wx)

            pltpu.sync_copy(px_ref.at[pl.ds(cbase, _K)], pxv)
            pltpu.sync_copy(py_ref.at[pl.ds(cbase, _K)], pyv)

            for i in range(_K // _L):
                sl = pl.ds(i * _L, _L)
                px = pxv[sl]
                py = pyv[sl]
                x0 = jnp.clip(px.astype(jnp.int32), 0, Wx - 1)
                x1 = jnp.minimum(x0 + 1, Wx - 1)
                y0 = jnp.clip(py.astype(jnp.int32), 0, Hx - 1)
                y1 = jnp.minimum(y0 + 1, Hx - 1)
                x0f = x0.astype(jnp.float32)
                x1f = x1.astype(jnp.float32)
                y0f = y0.astype(jnp.float32)
                y1f = y1.astype(jnp.float32)
                idx4[0, sl] = tb + y0 * Wx + x0
                idx4[1, sl] = tb + y1 * Wx + x0
                idx4[2, sl] = tb + y0 * Wx + x1
                idx4[3, sl] = tb + y1 * Wx + x1
                w4[0, sl] = (x1f - px) * (y1f - py)
                w4[1, sl] = (x1f - px) * (py - y0f)
                w4[2, sl] = (px - x0f) * (y1f - py)
                w4[3, sl] = (px - x0f) * (py - y0f)

            cps = [pltpu.async_copy(tbl_ref.at[idx4.at[q]], bufs.at[q], sem)
                   for q in range(4)]
            for cp in cps:
                cp.wait()

            def fma_body(g, _):
                gb = g * _L
                slg = pl.ds(gb, _L)
                wa16 = w4[0, slg]
                wb16 = w4[1, slg]
                wc16 = w4[2, slg]
                wd16 = w4[3, slg]
                for jj in range(_L):
                    j = gb + jj
                    wa = wa16[jj]
                    wb = wb16[jj]
                    wc = wc16[jj]
                    wd = wd16[jj]
                    for cg in range(C // _L):
                        slc = pl.ds(cg * _L, _L)
                        acc[j, slc] = ((wa * bufs[0, j, slc]
                                        + wb * bufs[1, j, slc])
                                       + wc * bufs[2, j, slc]) + wd * bufs[3, j, slc]
                return 0

            lax.fori_loop(0, _K // _L, fma_body, 0)
            pltpu.sync_copy(acc, out_ref.at[pl.ds(cbase, _K)])
            return carry

        lax.fori_loop(0, n_w, chunk_body, 0)

    return k(tbl, pxs, pys)


def kernel(x, pos, H, W):
    B, C, Hx, Wx = x.shape
    N = pos.shape[1]
    P = B * N
    tbl = x.transpose(0, 2, 3, 1).reshape(B * Hx * Wx, C)
    pxs = pos[..., 0].reshape(P) * (Wx - 1) / W
    pys = pos[..., 1].reshape(P) * (Hx - 1) / H
    out = _interp_sc(tbl, pxs, pys, B, C, Hx, Wx, N)
    return out.reshape(B, N, C)


# speedup vs baseline: 2048.2340x; 1.1618x over previous
import functools

import jax
import jax.numpy as jnp
from jax import lax
from jax.experimental import pallas as pl
from jax.experimental.pallas import tpu as pltpu
from jax.experimental.pallas import tpu_sc as plsc

_NC = 2
_NS = 16
_NW = _NC * _NS
_L = 16
_K = 80


def _interp_sc(tbl, posf, scl, B, C, Hx, Wx, N):
    P = B * N
    n_chunks = P // _K
    cpb = N // _K
    base = n_chunks // _NW
    rem = n_chunks % _NW
    maxw = base + 1

    mesh = plsc.VectorSubcoreMesh(
        core_axis_name="c", subcore_axis_name="s",
        num_cores=_NC, num_subcores=_NS)

    @functools.partial(
        pl.kernel,
        out_type=jax.ShapeDtypeStruct((P, C), jnp.float32),
        mesh=mesh,
        scratch_types=[
            pltpu.VMEM((maxw * 2 * _K,), jnp.float32),
            pltpu.VMEM((2, _L), jnp.float32),
            pltpu.VMEM((2, 4, _K), jnp.int32),
            pltpu.VMEM((2, 4, _K), jnp.float32),
            pltpu.VMEM((2, 4, _K, C), jnp.float32),
            pltpu.VMEM((2, _K, C), jnp.float32),
            pltpu.SemaphoreType.DMA,
            pltpu.SemaphoreType.DMA,
            pltpu.SemaphoreType.DMA,
            pltpu.SemaphoreType.DMA,
        ],
        compiler_params=pltpu.CompilerParams(use_tc_tiling_on_sc=False,
                                             needs_layout_passes=False),
    )
    def k(tbl_ref, pos_ref, scl_ref, out_ref, posb, sclv, idx4, w4, bufs, acc,
          gsem0, gsem1, osem0, osem1):
        gsem = (gsem0, gsem1)
        osem = (osem0, osem1)
        wid = lax.axis_index("s") * _NC + lax.axis_index("c")
        start = base * wid + jnp.minimum(wid, rem)
        n_w = base + (wid < rem).astype(jnp.int32)

        pltpu.sync_copy(scl_ref, sclv)
        pltpu.sync_copy(pos_ref.at[pl.ds(start * 2 * _K, maxw * 2 * _K)], posb)
        sxv = sclv[0, :]
        syv = sclv[1, :]
        lane2 = lax.broadcasted_iota(jnp.int32, (_L,), 0) * 2

        def stage_in(par, t):
            cid = start + t
            tb = (cid // cpb) * (Hx * Wx)
            tb0 = t * 2 * _K
            for i in range(_K // _L):
                sl = pl.ds(i * _L, _L)
                cols = lane2 + (tb0 + i * 2 * _L)
                px = plsc.load_gather(posb, [cols]) * sxv
                py = plsc.load_gather(posb, [cols + 1]) * syv
                x0 = jnp.clip(px.astype(jnp.int32), 0, Wx - 1)
                x1 = jnp.minimum(x0 + 1, Wx - 1)
                y0 = jnp.clip(py.astype(jnp.int32), 0, Hx - 1)
                y1 = jnp.minimum(y0 + 1, Hx - 1)
                x0f = x0.astype(jnp.float32)
                x1f = x1.astype(jnp.float32)
                y0f = y0.astype(jnp.float32)
                y1f = y1.astype(jnp.float32)
                idx4[par, 0, sl] = tb + y0 * Wx + x0
                idx4[par, 1, sl] = tb + y1 * Wx + x0
                idx4[par, 2, sl] = tb + y0 * Wx + x1
                idx4[par, 3, sl] = tb + y1 * Wx + x1
                w4[par, 0, sl] = (x1f - px) * (y1f - py)
                w4[par, 1, sl] = (x1f - px) * (py - y0f)
                w4[par, 2, sl] = (px - x0f) * (y1f - py)
                w4[par, 3, sl] = (px - x0f) * (py - y0f)
            for q in range(4):
                pltpu.async_copy(tbl_ref.at[idx4.at[par, q]],
                                 bufs.at[par, q], gsem[par])

        def stage_out(par, t):
            cid = start + t
            cbase = cid * _K
            for _ in range(4):
                pltpu.make_async_copy(tbl_ref.at[idx4.at[par, 0]],
                                      bufs.at[par, 0], gsem[par]).wait()

            @pl.when(t >= 2)
            def _():
                old = (cid - 2) * _K
                pltpu.make_async_copy(acc.at[par],
                                      out_ref.at[pl.ds(old, _K)],
                                      osem[par]).wait()

            def fma_body(g, _):
                gb = g * _L
                slg = pl.ds(gb, _L)
                wa16 = w4[par, 0, slg]
                wb16 = w4[par, 1, slg]
                wc16 = w4[par, 2, slg]
                wd16 = w4[par, 3, slg]
                for jj in range(_L):
                    j = gb + jj
                    wa = wa16[jj]
                    wb = wb16[jj]
                    wc = wc16[jj]
                    wd = wd16[jj]
                    for cg in range(C // _L):
                        slc = pl.ds(cg * _L, _L)
                        acc[par, j, slc] = (
                            (wa * bufs[par, 0, j, slc]
                             + wb * bufs[par, 1, j, slc])
                            + wc * bufs[par, 2, j, slc]) + wd * bufs[par, 3, j, slc]
                return 0

            lax.fori_loop(0, _K // _L, fma_body, 0)
            pltpu.async_copy(acc.at[par], out_ref.at[pl.ds(cbase, _K)],
                             osem[par])

        stage_in(0, 0)

        def pair_body(tt, carry):
            for par in range(2):
                t = tt * 2 + par

                @pl.when(t < n_w)
                def _():
                    @pl.when(t + 1 < n_w)
                    def _():
                        stage_in(1 - par, t + 1)

                    stage_out(par, t)
            return carry

        lax.fori_loop(0, (maxw + 1) // 2, pair_body, 0)

        for par in range(2):
            tl = n_w - 1 - ((n_w - 1 + par) & 1)
            old = (start + tl) * _K
            pltpu.make_async_copy(acc.at[par],
                                  out_ref.at[pl.ds(old, _K)],
                                  osem[par]).wait()

    return k(tbl, posf, scl)


def kernel(x, pos, H, W):
    B, C, Hx, Wx = x.shape
    N = pos.shape[1]
    P = B * N
    tbl = x.transpose(0, 2, 3, 1).reshape(B * Hx * Wx, C)
    posf = jnp.concatenate(
        [pos.reshape(P * 2), jnp.zeros((2 * _K,), jnp.float32)])
    sx = jnp.float32(Wx - 1) / W
    sy = jnp.float32(Hx - 1) / H
    scl = jnp.stack([jnp.full((_L,), sx, jnp.float32),
                     jnp.full((_L,), sy, jnp.float32)])
    out = _interp_sc(tbl, posf, scl, B, C, Hx, Wx, N)
    return out.reshape(B, N, C)
